# baseline (speedup 1.0000x reference)
import jax
import jax.numpy as jnp
from jax.experimental import pallas as pl
from jax.experimental.pallas import tpu as pltpu

_BT = 512


def _fused_body(s_ref, n_ref, W1_ref, W2_ref, b2_ref, W3_ref, W4_ref, b4_ref,
                out_ref):
    bt = s_ref.shape[0]
    ones2 = jnp.ones((2 * bt, 1), jnp.bfloat16)
    x = jnp.concatenate([s_ref[...], n_ref[...], ], axis=0)
    x1 = jnp.concatenate([x, ones2], axis=1)
    h = jnp.dot(x1, W1_ref[...], preferred_element_type=jnp.float32)
    hb = jnp.maximum(h, 0.0).astype(jnp.bfloat16)
    y = jnp.dot(hb, W2_ref[...], preferred_element_type=jnp.float32)
    yb = jnp.maximum(y + b2_ref[...], 0.0).astype(jnp.bfloat16)
    y2 = jnp.concatenate([yb[:bt], yb[bt:], jnp.ones((bt, 1), jnp.bfloat16)],
                         axis=1)
    h3 = jnp.dot(y2, W3_ref[...], preferred_element_type=jnp.float32)
    h3b = jnp.maximum(h3, 0.0).astype(jnp.bfloat16)
    out = jnp.dot(h3b, W4_ref[...], preferred_element_type=jnp.float32)
    out_ref[...] = out + b4_ref[...]


def kernel(state, next_state, W1, b1, W2, b2, W3, b3, W4, b4):
    B, sd = state.shape
    out_dim = W4.shape[1]
    grid = (B // _BT,)

    bf16 = jnp.bfloat16
    sb = state.astype(bf16)
    nb = next_state.astype(bf16)
    W1f = jnp.concatenate([W1, b1[None, :]], axis=0).astype(bf16)
    W3f = jnp.concatenate([W3, b3[None, :]], axis=0).astype(bf16)
    W2b = W2.astype(bf16)
    W4b = W4.astype(bf16)
    b2r = b2.reshape(1, -1)
    b4r = b4.reshape(1, -1)

    def _tile(i):
        return (i, 0)

    def _whole(i):
        return (0, 0)

    full = lambda a: pl.BlockSpec(a.shape, _whole)

    return pl.pallas_call(
        _fused_body,
        grid=grid,
        in_specs=[
            pl.BlockSpec((_BT, sd), _tile),
            pl.BlockSpec((_BT, sd), _tile),
            full(W1f), full(W2b), full(b2r), full(W3f), full(W4b), full(b4r),
        ],
        out_specs=pl.BlockSpec((_BT, out_dim), _tile),
        out_shape=jax.ShapeDtypeStruct((B, out_dim), jnp.float32),
        compiler_params=pltpu.CompilerParams(
            dimension_semantics=("arbitrary",),
            vmem_limit_bytes=100 * 1024 * 1024,
        ),
    )(sb, nb, W1f, W2b, b2r, W3f, W4b, b4r)
